# Initial kernel scaffold; baseline (speedup 1.0000x reference)
import jax
import jax.numpy as jnp
from jax.experimental import pallas as pl


def kernel(x, edge_index, adj_low_w, adj_high_w, Wl1, Wh1, Wm1, vl1, vh1, vm1, att1, Wl2, Wh2, Wm2, vl2, vh2, vm2, att2):
    raise NotImplementedError("write your pallas kernel here")


# speedup vs baseline: 4.3337x; 4.3337x over previous
import functools

import jax
import jax.numpy as jnp
from jax import lax
from jax.experimental import pallas as pl
from jax.experimental.pallas import tpu as pltpu
from jax.experimental.pallas import tpu_sc as plsc

NC = 2
NS = 16
LANES = 16
CHUNK = 128


def _mm_body(x_ref, w_ref, o1_ref, o2_ref, *, d2):
    h = jnp.dot(x_ref[...], w_ref[...], preferred_element_type=jnp.float32)
    o1_ref[...] = h[:, :d2]
    o2_ref[...] = h[:, d2:]


def _tc_matmul(x, wcat, d2, dm, bm=1000):
    n = x.shape[0]
    k = x.shape[1]
    grid = (n // bm,)
    return pl.pallas_call(
        functools.partial(_mm_body, d2=d2),
        grid=grid,
        in_specs=[
            pl.BlockSpec((bm, k), lambda i: (i, 0)),
            pl.BlockSpec((k, d2 + dm), lambda i: (0, 0)),
        ],
        out_specs=[
            pl.BlockSpec((bm, d2), lambda i: (i, 0)),
            pl.BlockSpec((bm, dm), lambda i: (i, 0)),
        ],
        out_shape=[
            jax.ShapeDtypeStruct((n, d2), jnp.float32),
            jax.ShapeDtypeStruct((n, dm), jnp.float32),
        ],
    )(x, wcat)


def _make_sc_spmm(npad, epad, d2):
    dh = d2 // 2
    epc = epad // NC
    epw = epc // NS
    nchunks = epw // CHUNK
    rows_ps = npad // NS

    mesh = plsc.VectorSubcoreMesh(core_axis_name="c", subcore_axis_name="s")

    @functools.partial(
        pl.kernel,
        mesh=mesh,
        out_type=jax.ShapeDtypeStruct((NC, npad, d2), jnp.float32),
        scratch_types=[
            pltpu.VMEM((CHUNK,), jnp.int32),
            pltpu.VMEM((CHUNK,), jnp.int32),
            pltpu.VMEM((CHUNK,), jnp.float32),
            pltpu.VMEM((CHUNK,), jnp.float32),
            pltpu.VMEM((CHUNK, d2), jnp.float32),
            pltpu.VMEM_SHARED((npad, d2), jnp.float32),
            pltpu.SemaphoreType.DMA,
        ],
    )
    def sc_kernel(h_hbm, src_hbm, dst_hbm, wl_hbm, wh_hbm, z_hbm, out_hbm,
                  src_v, dst_v, wl_v, wh_v, rows_v, acc_sh, sem):
        c = lax.axis_index("c")
        s = lax.axis_index("s")
        r0 = s * rows_ps
        pltpu.sync_copy(z_hbm.at[pl.ds(r0, rows_ps)], acc_sh.at[pl.ds(r0, rows_ps)])
        plsc.subcore_barrier()

        base = c * epc + s * epw

        def chunk_body(i, carry):
            eb = base + i * CHUNK
            pltpu.sync_copy(src_hbm.at[pl.ds(eb, CHUNK)], src_v)
            pltpu.sync_copy(dst_hbm.at[pl.ds(eb, CHUNK)], dst_v)
            pltpu.sync_copy(wl_hbm.at[pl.ds(eb, CHUNK)], wl_v)
            pltpu.sync_copy(wh_hbm.at[pl.ds(eb, CHUNK)], wh_v)
            pltpu.async_copy(h_hbm.at[src_v], rows_v, sem).wait()

            def grp_body(g, carry2):
                e0 = g * LANES
                w16l = wl_v[pl.ds(e0, LANES)]
                w16h = wh_v[pl.ds(e0, LANES)]
                for l in range(LANES):
                    for j in range(d2 // LANES):
                        sp = w16l[l] if j < dh // LANES else w16h[l]
                        rows_v[e0 + l, pl.ds(j * LANES, LANES)] = (
                            rows_v[e0 + l, pl.ds(j * LANES, LANES)] * sp)
                return carry2

            lax.fori_loop(0, CHUNK // LANES, grp_body, 0)
            pltpu.sync_copy(rows_v, acc_sh.at[dst_v], add=True)
            return carry

        lax.fori_loop(0, nchunks, chunk_body, 0)
        plsc.subcore_barrier()
        pltpu.sync_copy(acc_sh.at[pl.ds(r0, rows_ps)],
                        out_hbm.at[c].at[pl.ds(r0, rows_ps)])

    return sc_kernel


def _combine_body(p_ref, hm_ref, vl_ref, vh_ref, vm_ref, att_ref, *outs, dm, dpad,
                  final):
    low = jax.nn.relu(p_ref[0, :, 0:dm] + p_ref[1, :, 0:dm])
    high = jax.nn.relu(p_ref[0, :, dpad:dpad + dm] + p_ref[1, :, dpad:dpad + dm])
    mlp = jax.nn.relu(hm_ref[...])
    sl = jnp.sum(low * vl_ref[...], axis=1, keepdims=True)
    sh = jnp.sum(high * vh_ref[...], axis=1, keepdims=True)
    sm = jnp.sum(mlp * vm_ref[...], axis=1, keepdims=True)
    gl = jax.nn.sigmoid(sl)
    gh = jax.nn.sigmoid(sh)
    gm = jax.nn.sigmoid(sm)
    a = [(gl * att_ref[0, j] + gh * att_ref[1, j] + gm * att_ref[2, j]) / 3.0
         for j in range(3)]
    m = jnp.maximum(jnp.maximum(a[0], a[1]), a[2])
    e = [jnp.exp(aj - m) for aj in a]
    tot = e[0] + e[1] + e[2]
    out = (3.0 / tot) * (e[0] * low + e[1] * high + e[2] * mlp)
    if final:
        mx = jnp.max(out, axis=1, keepdims=True)
        ex = jnp.exp(out - mx)
        ss = jnp.sum(ex, axis=1, keepdims=True)
        outs[0][...] = out - mx - jnp.log(ss)
        outs[1][...] = ex / ss
    else:
        outs[0][...] = jax.nn.relu(out)


def _tc_combine(p, hm, vl, vh, vm, att, dm, dpad, final, bm=1000):
    n = hm.shape[0]
    d2 = p.shape[2]
    grid = (n // bm,)
    if final:
        out_shape = [jax.ShapeDtypeStruct((n, dm), jnp.float32)] * 2
        out_specs = [pl.BlockSpec((bm, dm), lambda i: (i, 0))] * 2
    else:
        out_shape = [jax.ShapeDtypeStruct((n, dm), jnp.float32)]
        out_specs = [pl.BlockSpec((bm, dm), lambda i: (i, 0))]
    res = pl.pallas_call(
        functools.partial(_combine_body, dm=dm, dpad=dpad, final=final),
        grid=grid,
        in_specs=[
            pl.BlockSpec((2, bm, d2), lambda i: (0, i, 0)),
            pl.BlockSpec((bm, dm), lambda i: (i, 0)),
            pl.BlockSpec((1, dm), lambda i: (0, 0)),
            pl.BlockSpec((1, dm), lambda i: (0, 0)),
            pl.BlockSpec((1, dm), lambda i: (0, 0)),
            pl.BlockSpec(memory_space=pltpu.SMEM),
        ],
        out_specs=out_specs,
        out_shape=out_shape,
    )(p, hm, vl, vh, vm, att)
    return res


def _layer(xin, src, dst, wl, wh, Wl, Wh, Wm, vl, vh, vm, att, dpad, npad, epad,
           zeros, final):
    din, dm = Wl.shape
    pad = dpad - dm
    if pad:
        Wlp = jnp.pad(Wl, ((0, 0), (0, pad)))
        Whp = jnp.pad(Wh, ((0, 0), (0, pad)))
    else:
        Wlp, Whp = Wl, Wh
    wcat = jnp.concatenate([Wlp, Whp, Wm], axis=1)
    d2 = 2 * dpad
    hcat, hm = _tc_matmul(xin, wcat, d2, dm)
    spmm = _make_sc_spmm(npad, epad, d2)
    parts = spmm(hcat, src, dst, wl, wh, zeros)
    return _tc_combine(parts, hm, vl.reshape(1, dm), vh.reshape(1, dm),
                       vm.reshape(1, dm), att, dm, dpad, final)


def kernel(x, edge_index, adj_low_w, adj_high_w, Wl1, Wh1, Wm1, vl1, vh1, vm1,
           att1, Wl2, Wh2, Wm2, vl2, vh2, vm2, att2):
    n = x.shape[0]
    e = edge_index.shape[1]
    npad = ((n + NS * 8 - 1) // (NS * 8)) * NS * 8
    estep = NC * NS * CHUNK
    epad = ((e + estep - 1) // estep) * estep
    dst = edge_index[0]
    src = edge_index[1]
    if epad != e:
        pad = epad - e
        dst = jnp.pad(dst, (0, pad))
        src = jnp.pad(src, (0, pad))
        adj_low_w = jnp.pad(adj_low_w, (0, pad))
        adj_high_w = jnp.pad(adj_high_w, (0, pad))

    z = jnp.zeros((npad, 128), jnp.float32)
    fea = _layer(x, src, dst, adj_low_w, adj_high_w, Wl1, Wh1, Wm1, vl1, vh1,
                 vm1, att1, 64, npad, epad, z, final=False)[0]
    lsm, sm = _layer(fea, src, dst, adj_low_w, adj_high_w, Wl2, Wh2, Wm2, vl2,
                     vh2, vm2, att2, 64, npad, epad, z, final=True)
    return (lsm, sm)
